# baseline (speedup 1.0000x reference)
import functools

import jax
import jax.numpy as jnp
from jax import lax
from jax.experimental import pallas as pl
from jax.experimental.pallas import tpu as pltpu
from jax.experimental.pallas import tpu_sc as plsc

F32 = jnp.float32
BF16 = jnp.bfloat16
I32 = jnp.int32

NC = 2
NS = 16
NW = NC * NS
_SC_PARAMS = pltpu.CompilerParams(needs_layout_passes=False)


def _gelu(v):
    return 0.5 * v * (1.0 + lax.erf(v * (2.0 ** -0.5)))


def _make_win(E, NP):
    EW = E // NS
    NGRP = EW // 16
    CW = NP // NS
    mesh = plsc.VectorSubcoreMesh(core_axis_name="c", subcore_axis_name="s")

    @functools.partial(
        pl.kernel,
        out_type=(
            jax.ShapeDtypeStruct((NP,), I32),
            jax.ShapeDtypeStruct((NP,), I32),
        ),
        mesh=mesh,
        compiler_params=_SC_PARAMS,
        scratch_types=[
            pltpu.VMEM((NP,), I32),
            pltpu.VMEM((EW,), I32),
            pltpu.VMEM((NS * CW,), I32),
            pltpu.VMEM_SHARED((NS, NP), I32),
        ],
    )
    def win_k(dst_hbm, win_out, wing_out, winb, dstb, wtbl, win_sh):
        c = lax.axis_index("c")
        sid = lax.axis_index("s")
        iota = lax.iota(I32, 16)

        @pl.when(c == 0)
        def _():
            base = sid * EW
            pltpu.sync_copy(dst_hbm.at[pl.ds(base, EW)], dstb)
            neg1 = jnp.full((16,), -1, I32)

            def initw(i, carry):
                winb[pl.ds(i * 16, 16)] = neg1
                return carry

            lax.fori_loop(0, NP // 16, initw, 0)

            def wing(g, carry):
                dv = dstb[pl.ds(g * 16, 16)]
                ids = iota + (base + g * 16)
                for j in range(16):
                    plsc.store_scatter(winb, [dv], ids, mask=(iota == j))
                return carry

            lax.fori_loop(0, NGRP, wing, 0)
            pltpu.sync_copy(winb, win_sh.at[sid])
            plsc.subcore_barrier()

            cb = sid * CW

            def ld(t, carry):
                pltpu.sync_copy(win_sh.at[t, pl.ds(cb, CW)],
                                wtbl.at[pl.ds(t * CW, CW)])
                return carry

            lax.fori_loop(0, NS, ld, 0)

            def mrg(i, carry):
                def red(t, acc):
                    return jnp.maximum(acc, wtbl[pl.ds(t * CW + i * 16, 16)])

                mv = lax.fori_loop(0, NS, red, jnp.full((16,), -1, I32))
                sl = pl.ds(i * 16, 16)
                winb[sl] = mv
                dstb[sl] = lax.shift_right_logical(jnp.maximum(mv, 0), 4)
                return carry

            lax.fori_loop(0, CW // 16, mrg, 0)
            pltpu.sync_copy(winb.at[pl.ds(0, CW)], win_out.at[pl.ds(cb, CW)])
            pltpu.sync_copy(dstb.at[pl.ds(0, CW)], wing_out.at[pl.ds(cb, CW)])

    return win_k


def _make_gather(N, E, D2, GCH, NB):
    EW = E // NW
    NI = EW // GCH
    NGR = NI // NB
    mesh = plsc.VectorSubcoreMesh(core_axis_name="c", subcore_axis_name="s")

    @functools.partial(
        pl.kernel,
        out_type=(
            jax.ShapeDtypeStruct((E, D2), F32),
            jax.ShapeDtypeStruct((E, D2), F32),
        ),
        mesh=mesh,
        compiler_params=_SC_PARAMS,
        scratch_types=[
            pltpu.VMEM((EW,), I32),
            pltpu.VMEM((EW,), I32),
            pltpu.VMEM((NB, GCH, D2), F32),
            pltpu.VMEM((NB, GCH, D2), F32),
            pltpu.VMEM_SHARED((N, D2), F32),
        ] + [pltpu.SemaphoreType.DMA] * (2 * NB),
    )
    def gather_k(x_hbm, src_hbm, dst_hbm, s_out, d_out,
                 srcb, dstb, rows_s, rows_d, x_sh, *sems):
        gsem = sems[:NB]
        wsem = sems[NB:]
        c = lax.axis_index("c")
        sid = lax.axis_index("s")
        base = (c * NS + sid) * EW

        @pl.when(sid < 10)
        def _():
            pltpu.sync_copy(x_hbm.at[pl.ds(sid * (N // 10), N // 10)],
                            x_sh.at[pl.ds(sid * (N // 10), N // 10)])

        pltpu.sync_copy(src_hbm.at[pl.ds(base, EW)], srcb)
        pltpu.sync_copy(dst_hbm.at[pl.ds(base, EW)], dstb)
        plsc.subcore_barrier()

        def gath(i, b):
            off = i * GCH
            pltpu.async_copy(
                x_sh.at[srcb.at[pl.ds(off, GCH)]], rows_s.at[b], gsem[b])
            pltpu.async_copy(
                x_sh.at[dstb.at[pl.ds(off, GCH)]], rows_d.at[b], gsem[b])

        def drain(i, b):
            off = i * GCH
            pltpu.make_async_copy(
                x_sh.at[srcb.at[pl.ds(off, GCH)]], rows_s.at[b],
                gsem[b]).wait()
            pltpu.make_async_copy(
                x_sh.at[dstb.at[pl.ds(off, GCH)]], rows_d.at[b],
                gsem[b]).wait()
            so = s_out.at[pl.ds(base + off, GCH)]
            do = d_out.at[pl.ds(base + off, GCH)]
            pltpu.async_copy(rows_s.at[b], so, wsem[b])
            pltpu.async_copy(rows_d.at[b], do, wsem[b])
            pltpu.make_async_copy(rows_s.at[b], so, wsem[b]).wait()
            pltpu.make_async_copy(rows_d.at[b], do, wsem[b]).wait()

        for b in range(NB):
            gath(b, b)

        def body(g, carry):
            for b in range(NB):
                i = g * NB + b
                drain(i, b)

                @pl.when(g < NGR - 1)
                def _():
                    gath(i + NB, b)

            return carry

        lax.fori_loop(0, NGR, body, 0)
        for r in range(NGR * NB, NI):
            gath(r, r % NB)
            drain(r, r % NB)

    return gather_k


def _edge_body(s_ref, d_ref, ea_ref, w1s, w1d, w1e, b1, w2, b2,
               wq, bq, wkvs, wkve, bkv, sel, m_ref, ap_ref):
    s = s_ref[...].astype(BF16)
    d = d_ref[...].astype(BF16)
    ea = ea_ref[...].astype(BF16)
    h1 = jnp.dot(s, w1s[...], preferred_element_type=F32)
    h1 = h1 + jnp.dot(d, w1d[...], preferred_element_type=F32)
    h1 = h1 + jnp.dot(ea, w1e[...], preferred_element_type=F32) + b1[...]
    m_ref[...] = jnp.dot(_gelu(h1), w2[...].astype(BF16),
                         preferred_element_type=F32) + b2[...]
    q = jnp.dot(d, wq[...], preferred_element_type=F32) + bq[...]
    kv = (jnp.dot(s, wkvs[...], preferred_element_type=F32)
          + jnp.dot(ea, wkve[...], preferred_element_type=F32) + bkv[...])
    qk = q * kv
    be = qk.shape[0]
    qk3 = qk.reshape(be // 16, 16, qk.shape[1])
    for r in range(16):
        a_r = jnp.dot(qk3[:, r, :], sel[...], preferred_element_type=F32)
        ap_ref[:, r * 8:(r + 1) * 8] = a_r


def _make_edge(E, D, ED, H, BE):
    full = lambda shape: pl.BlockSpec(shape, lambda i: (0, 0))
    return pl.pallas_call(
        _edge_body,
        grid=(E // BE,),
        in_specs=[
            pl.BlockSpec((BE, D), lambda i: (i, 0)),
            pl.BlockSpec((BE, D), lambda i: (i, 0)),
            pl.BlockSpec((BE, ED), lambda i: (i, 0)),
            full((D, D)), full((D, D)), full((ED, D)), full((1, D)),
            full((D, D)), full((1, D)),
            full((D, D)), full((1, D)),
            full((D, D)), full((ED, D)), full((1, D)),
            full((D, H)),
        ],
        out_specs=[
            pl.BlockSpec((BE, D), lambda i: (i, 0)),
            pl.BlockSpec((BE // 16, 16 * H), lambda i: (i, 0)),
        ],
        out_shape=[
            jax.ShapeDtypeStruct((E, D), F32),
            jax.ShapeDtypeStruct((E // 16, 16 * H), F32),
        ],
    )


def _make_select(E, H, NP, GCH):
    CW = NP // NW
    mesh = plsc.VectorSubcoreMesh(core_axis_name="c", subcore_axis_name="s")

    @functools.partial(
        pl.kernel,
        out_type=jax.ShapeDtypeStruct((NP * H,), F32),
        mesh=mesh,
        compiler_params=_SC_PARAMS,
        scratch_types=[
            pltpu.VMEM((CW,), I32),
            pltpu.VMEM((CW,), I32),
            pltpu.VMEM((CW, 16 * H), F32),
            pltpu.VMEM((CW * H,), F32),
            pltpu.SemaphoreType.DMA,
        ],
    )
    def select_k(win_hbm, wing_hbm, ap_hbm, af_out,
                 winb, wgb, rows, aout, sem):
        c = lax.axis_index("c")
        sid = lax.axis_index("s")
        w = c * NS + sid
        wbase = w * CW
        iota = lax.iota(I32, 16)

        pltpu.sync_copy(win_hbm.at[pl.ds(wbase, CW)], winb)
        pltpu.sync_copy(wing_hbm.at[pl.ds(wbase, CW)], wgb)
        for k in range(CW // GCH):
            sl = pl.ds(k * GCH, GCH)
            pltpu.async_copy(ap_hbm.at[wgb.at[sl]], rows.at[sl], sem).wait()

        neg = jnp.full((16,), -1e9, F32)

        def selp(i, carry):
            v = i * 16 + iota
            widx = lax.shift_right_logical(v, 3)
            h = lax.bitwise_and(v, 7)
            wv = plsc.load_gather(winb, [widx])
            rsub = lax.bitwise_and(jnp.maximum(wv, 0), 15)
            av = plsc.load_gather(rows, [widx, rsub * 8 + h])
            aout[pl.ds(i * 16, 16)] = jnp.where(wv >= 0, av, neg)
            return carry

        lax.fori_loop(0, (CW * H) // 16, selp, 0)
        pltpu.sync_copy(aout, af_out.at[pl.ds(wbase * H, CW * H)])

    return select_k


def _make_scatter(E, D, NP, SCH, NB):
    EW = E // NW
    RPW = NP // NS
    NSC = EW // SCH
    NGR = NSC // NB
    mesh = plsc.VectorSubcoreMesh(core_axis_name="c", subcore_axis_name="s")

    @functools.partial(
        pl.kernel,
        out_type=jax.ShapeDtypeStruct((NC, NP, D), F32),
        mesh=mesh,
        compiler_params=_SC_PARAMS,
        scratch_types=[
            pltpu.VMEM((NB, SCH), I32),
            pltpu.VMEM((NB, SCH, D), F32),
            pltpu.VMEM_SHARED((NP, D), F32),
        ] + [pltpu.SemaphoreType.DMA] * NB,
    )
    def scatter_k(dst_hbm, m_hbm, zeros_hbm, msum2, idxb, mrows, msum_sh,
                  *lsem):
        c = lax.axis_index("c")
        sid = lax.axis_index("s")
        base = (c * NS + sid) * EW

        pltpu.sync_copy(zeros_hbm.at[pl.ds(sid * RPW, RPW)],
                        msum_sh.at[pl.ds(sid * RPW, RPW)])
        plsc.subcore_barrier()

        def load(i, b):
            off = base + i * SCH
            pltpu.async_copy(dst_hbm.at[pl.ds(off, SCH)], idxb.at[b],
                             lsem[b])
            pltpu.async_copy(m_hbm.at[pl.ds(off, SCH)], mrows.at[b],
                             lsem[b])

        for b in range(NB):
            load(b, b)

        def body(g, carry):
            for b in range(NB):
                i = g * NB + b
                off = base + i * SCH
                pltpu.make_async_copy(dst_hbm.at[pl.ds(off, SCH)],
                                      idxb.at[b], lsem[b]).wait()
                pltpu.make_async_copy(m_hbm.at[pl.ds(off, SCH)],
                                      mrows.at[b], lsem[b]).wait()
                pltpu.sync_copy(mrows.at[b], msum_sh.at[idxb.at[b]],
                                add=True)

                @pl.when(g < NGR - 1)
                def _():
                    load(i + NB, b)

            return carry

        lax.fori_loop(0, NGR, body, 0)
        plsc.subcore_barrier()
        pltpu.sync_copy(msum_sh.at[pl.ds(sid * RPW, RPW)],
                        msum2.at[c, pl.ds(sid * RPW, RPW)])

    return scatter_k


def _soft_body(a_ref, p_ref):
    a = a_ref[...]
    mx = jnp.max(a, axis=0, keepdims=True)
    e = jnp.exp(a - mx)
    p_ref[...] = e / jnp.sum(e, axis=0, keepdims=True)


def _out_body(x_ref, m0_ref, m1_ref, m2_ref, m3_ref, p_ref, wo1x, wo1a,
              bo1, wo2, bo2, lng, lnb, rep, y_ref):
    x = x_ref[...]
    pfull = jnp.dot(p_ref[...], rep[...], preferred_element_type=F32)
    agg = ((m0_ref[...] + m1_ref[...])
           + (m2_ref[...] + m3_ref[...])) * pfull
    h1 = (jnp.dot(x, wo1x[...], preferred_element_type=F32)
          + jnp.dot(agg, wo1a[...], preferred_element_type=F32) + bo1[...])
    h = jnp.dot(_gelu(h1), wo2[...], preferred_element_type=F32) + bo2[...]
    y = x + h
    mu = jnp.mean(y, axis=1, keepdims=True)
    var = jnp.mean((y - mu) ** 2, axis=1, keepdims=True)
    y_ref[...] = (y - mu) / jnp.sqrt(var + 1e-5) * lng[...] + lnb[...]


def _make_out(N, D, H, BN):
    full = lambda shape: pl.BlockSpec(shape, lambda i: (0, 0))
    return pl.pallas_call(
        _out_body,
        grid=(N // BN,),
        in_specs=[
            pl.BlockSpec((BN, D), lambda i: (i, 0)),
            pl.BlockSpec((BN, D), lambda i: (i, 0)),
            pl.BlockSpec((BN, D), lambda i: (i, 0)),
            pl.BlockSpec((BN, D), lambda i: (i, 0)),
            pl.BlockSpec((BN, D), lambda i: (i, 0)),
            pl.BlockSpec((BN, H), lambda i: (i, 0)),
            full((D, D)), full((D, D)), full((1, D)),
            full((D, D)), full((1, D)),
            full((1, D)), full((1, D)),
            full((H, D)),
        ],
        out_specs=pl.BlockSpec((BN, D), lambda i: (i, 0)),
        out_shape=jax.ShapeDtypeStruct((N, D), F32),
    )


def kernel(node_features, edge_index, edge_attr, W_msg1, b_msg1, W_msg2,
           b_msg2, W_q, b_q, W_kv, b_kv, W_out1, b_out1, W_out2, b_out2,
           ln_g, ln_b):
    x = node_features
    N, D = x.shape
    E = edge_index.shape[1]
    ED = edge_attr.shape[1]
    H = 8
    HD = D // H
    NP = ((N + 32 * 8 - 1) // (32 * 8)) * (32 * 8)

    src = edge_index[0]
    dst = edge_index[1]

    w1s = W_msg1[:D]
    w1d = W_msg1[D:2 * D]
    w1e = W_msg1[2 * D:]
    wkvs = W_kv[:D]
    wkve = W_kv[D:]
    wo1x = W_out1[:D]
    wo1a = W_out1[D:]
    b1 = b_msg1.reshape(1, D)
    b2 = b_msg2.reshape(1, D)
    bq = b_q.reshape(1, D)
    bkv = b_kv.reshape(1, D)
    bo1 = b_out1.reshape(1, D)
    bo2 = b_out2.reshape(1, D)
    lng = ln_g.reshape(1, D)
    lnb = ln_b.reshape(1, D)

    head_of = jnp.arange(D, dtype=I32) // HD
    sel = (head_of[:, None] == jnp.arange(H, dtype=I32)[None, :]
           ).astype(F32) * (HD ** -0.5)
    rep = (jnp.arange(H, dtype=I32)[:, None] == head_of[None, :]).astype(F32)
    zeros = jnp.zeros((NP, D), F32)

    win, wing = _make_win(E, NP)(dst)
    EH = E // 2
    gat = _make_gather(N, EH, D, 40, 2)
    edg = _make_edge(EH, D, ED, H, 2560)
    sca = _make_scatter(EH, D, NP, 40, 5)
    wb = (w1s.astype(BF16), w1d.astype(BF16), w1e.astype(BF16), b1,
          W_msg2, b2, W_q.astype(BF16), bq, wkvs.astype(BF16),
          wkve.astype(BF16), bkv, sel)
    s1, d1 = gat(x, src[:EH], dst[:EH])
    m1, ap1 = edg(s1, d1, edge_attr[:EH], *wb)
    s2, d2 = gat(x, src[EH:], dst[EH:])
    m2, ap2 = edg(s2, d2, edge_attr[EH:], *wb)
    msA = sca(dst[:EH], m1, zeros)
    msB = sca(dst[EH:], m2, zeros)
    ap = jnp.concatenate([ap1, ap2], axis=0)
    af_flat = _make_select(E, H, NP, 80)(win, wing, ap)
    p = pl.pallas_call(
        _soft_body, out_shape=jax.ShapeDtypeStruct((NP, H), F32))(
            af_flat.reshape(NP, H))
    y = _make_out(N, D, H, 1000)(
        x, msA[0], msA[1], msB[0], msB[1], p, wo1x, wo1a, bo1, W_out2, bo2,
        lng, lnb, rep)
    return y


# speedup vs baseline: 1.0847x; 1.0847x over previous
import functools

import jax
import jax.numpy as jnp
from jax import lax
from jax.experimental import pallas as pl
from jax.experimental.pallas import tpu as pltpu
from jax.experimental.pallas import tpu_sc as plsc

F32 = jnp.float32
BF16 = jnp.bfloat16
I32 = jnp.int32

NC = 2
NS = 16
NW = NC * NS
_SC_PARAMS = pltpu.CompilerParams(needs_layout_passes=False)


def _gelu(v):
    return 0.5 * v * (1.0 + lax.erf(v * (2.0 ** -0.5)))


def _make_win(E, NP):
    EW = E // NS
    NGRP = EW // 16
    CW = NP // NS
    mesh = plsc.VectorSubcoreMesh(core_axis_name="c", subcore_axis_name="s")

    @functools.partial(
        pl.kernel,
        out_type=(
            jax.ShapeDtypeStruct((NP,), I32),
            jax.ShapeDtypeStruct((NP,), I32),
        ),
        mesh=mesh,
        compiler_params=_SC_PARAMS,
        scratch_types=[
            pltpu.VMEM((NP,), I32),
            pltpu.VMEM((EW,), I32),
            pltpu.VMEM((NS * CW,), I32),
            pltpu.VMEM_SHARED((NS, NP), I32),
        ],
    )
    def win_k(dst_hbm, win_out, wing_out, winb, dstb, wtbl, win_sh):
        c = lax.axis_index("c")
        sid = lax.axis_index("s")
        iota = lax.iota(I32, 16)

        @pl.when(c == 0)
        def _():
            base = sid * EW
            pltpu.sync_copy(dst_hbm.at[pl.ds(base, EW)], dstb)
            neg1 = jnp.full((16,), -1, I32)

            def initw(i, carry):
                winb[pl.ds(i * 16, 16)] = neg1
                return carry

            lax.fori_loop(0, NP // 16, initw, 0)

            def wing(g, carry):
                dv = dstb[pl.ds(g * 16, 16)]
                ids = iota + (base + g * 16)
                for j in range(16):
                    plsc.store_scatter(winb, [dv], ids, mask=(iota == j))
                return carry

            lax.fori_loop(0, NGRP, wing, 0)
            pltpu.sync_copy(winb, win_sh.at[sid])
            plsc.subcore_barrier()

            cb = sid * CW

            def ld(t, carry):
                pltpu.sync_copy(win_sh.at[t, pl.ds(cb, CW)],
                                wtbl.at[pl.ds(t * CW, CW)])
                return carry

            lax.fori_loop(0, NS, ld, 0)

            def mrg(i, carry):
                def red(t, acc):
                    return jnp.maximum(acc, wtbl[pl.ds(t * CW + i * 16, 16)])

                mv = lax.fori_loop(0, NS, red, jnp.full((16,), -1, I32))
                sl = pl.ds(i * 16, 16)
                winb[sl] = mv
                dstb[sl] = lax.shift_right_logical(jnp.maximum(mv, 0), 4)
                return carry

            lax.fori_loop(0, CW // 16, mrg, 0)
            pltpu.sync_copy(winb.at[pl.ds(0, CW)], win_out.at[pl.ds(cb, CW)])
            pltpu.sync_copy(dstb.at[pl.ds(0, CW)], wing_out.at[pl.ds(cb, CW)])

    return win_k


def _make_gather(N, E, D2, GCH, NB):
    EW = E // NW
    NI = EW // GCH
    NGR = NI // NB
    mesh = plsc.VectorSubcoreMesh(core_axis_name="c", subcore_axis_name="s")

    @functools.partial(
        pl.kernel,
        out_type=(
            jax.ShapeDtypeStruct((E, D2), F32),
            jax.ShapeDtypeStruct((E, D2), F32),
        ),
        mesh=mesh,
        compiler_params=_SC_PARAMS,
        scratch_types=[
            pltpu.VMEM((EW,), I32),
            pltpu.VMEM((EW,), I32),
            pltpu.VMEM((NB, GCH, D2), F32),
            pltpu.VMEM((NB, GCH, D2), F32),
            pltpu.VMEM_SHARED((N, D2), F32),
        ] + [pltpu.SemaphoreType.DMA] * (2 * NB),
    )
    def gather_k(x_hbm, src_hbm, dst_hbm, s_out, d_out,
                 srcb, dstb, rows_s, rows_d, x_sh, *sems):
        gsem = sems[:NB]
        wsem = sems[NB:]
        c = lax.axis_index("c")
        sid = lax.axis_index("s")
        base = (c * NS + sid) * EW

        @pl.when(sid < 10)
        def _():
            pltpu.sync_copy(x_hbm.at[pl.ds(sid * (N // 10), N // 10)],
                            x_sh.at[pl.ds(sid * (N // 10), N // 10)])

        pltpu.sync_copy(src_hbm.at[pl.ds(base, EW)], srcb)
        pltpu.sync_copy(dst_hbm.at[pl.ds(base, EW)], dstb)
        plsc.subcore_barrier()

        def gath(i, b):
            off = i * GCH
            pltpu.async_copy(
                x_sh.at[srcb.at[pl.ds(off, GCH)]], rows_s.at[b], gsem[b])
            pltpu.async_copy(
                x_sh.at[dstb.at[pl.ds(off, GCH)]], rows_d.at[b], gsem[b])

        def drain(i, b):
            off = i * GCH
            pltpu.make_async_copy(
                x_sh.at[srcb.at[pl.ds(off, GCH)]], rows_s.at[b],
                gsem[b]).wait()
            pltpu.make_async_copy(
                x_sh.at[dstb.at[pl.ds(off, GCH)]], rows_d.at[b],
                gsem[b]).wait()
            so = s_out.at[pl.ds(base + off, GCH)]
            do = d_out.at[pl.ds(base + off, GCH)]
            pltpu.async_copy(rows_s.at[b], so, wsem[b])
            pltpu.async_copy(rows_d.at[b], do, wsem[b])
            pltpu.make_async_copy(rows_s.at[b], so, wsem[b]).wait()
            pltpu.make_async_copy(rows_d.at[b], do, wsem[b]).wait()

        for b in range(NB):
            gath(b, b)

        def body(g, carry):
            for b in range(NB):
                i = g * NB + b
                drain(i, b)

                @pl.when(g < NGR - 1)
                def _():
                    gath(i + NB, b)

            return carry

        lax.fori_loop(0, NGR, body, 0)
        for r in range(NGR * NB, NI):
            gath(r, r % NB)
            drain(r, r % NB)

    return gather_k


def _edge_body(s_ref, d_ref, ea_ref, w1s, w1d, w1e, b1, w2, b2,
               wq, bq, wkvs, wkve, bkv, sel, m_ref, ap_ref):
    s = s_ref[...].astype(BF16)
    d = d_ref[...].astype(BF16)
    ea = ea_ref[...].astype(BF16)
    h1 = jnp.dot(s, w1s[...], preferred_element_type=F32)
    h1 = h1 + jnp.dot(d, w1d[...], preferred_element_type=F32)
    h1 = h1 + jnp.dot(ea, w1e[...], preferred_element_type=F32) + b1[...]
    m_ref[...] = jnp.dot(_gelu(h1), w2[...].astype(BF16),
                         preferred_element_type=F32) + b2[...]
    q = jnp.dot(d, wq[...], preferred_element_type=F32) + bq[...]
    kv = (jnp.dot(s, wkvs[...], preferred_element_type=F32)
          + jnp.dot(ea, wkve[...], preferred_element_type=F32) + bkv[...])
    qk = (q * kv).astype(BF16)
    be = qk.shape[0]
    qkg = qk.reshape(be // 16, 16 * qk.shape[1])
    ap_ref[...] = jnp.dot(qkg, sel[...], preferred_element_type=F32)


def _make_edge(E, D, ED, H, BE):
    full = lambda shape: pl.BlockSpec(shape, lambda i: (0, 0))
    return pl.pallas_call(
        _edge_body,
        grid=(E // BE,),
        in_specs=[
            pl.BlockSpec((BE, D), lambda i: (i, 0)),
            pl.BlockSpec((BE, D), lambda i: (i, 0)),
            pl.BlockSpec((BE, ED), lambda i: (i, 0)),
            full((D, D)), full((D, D)), full((ED, D)), full((1, D)),
            full((D, D)), full((1, D)),
            full((D, D)), full((1, D)),
            full((D, D)), full((ED, D)), full((1, D)),
            full((16 * D, 16 * H)),
        ],
        out_specs=[
            pl.BlockSpec((BE, D), lambda i: (i, 0)),
            pl.BlockSpec((BE // 16, 16 * H), lambda i: (i, 0)),
        ],
        out_shape=[
            jax.ShapeDtypeStruct((E, D), F32),
            jax.ShapeDtypeStruct((E // 16, 16 * H), F32),
        ],
    )


def _make_select(E, H, NP, GCH):
    CW = NP // NW
    mesh = plsc.VectorSubcoreMesh(core_axis_name="c", subcore_axis_name="s")

    @functools.partial(
        pl.kernel,
        out_type=jax.ShapeDtypeStruct((NP * H,), F32),
        mesh=mesh,
        compiler_params=_SC_PARAMS,
        scratch_types=[
            pltpu.VMEM((CW,), I32),
            pltpu.VMEM((CW,), I32),
            pltpu.VMEM((CW, 16 * H), F32),
            pltpu.VMEM((CW * H,), F32),
            pltpu.SemaphoreType.DMA,
        ],
    )
    def select_k(win_hbm, wing_hbm, ap_hbm, af_out,
                 winb, wgb, rows, aout, sem):
        c = lax.axis_index("c")
        sid = lax.axis_index("s")
        w = c * NS + sid
        wbase = w * CW
        iota = lax.iota(I32, 16)

        pltpu.sync_copy(win_hbm.at[pl.ds(wbase, CW)], winb)
        pltpu.sync_copy(wing_hbm.at[pl.ds(wbase, CW)], wgb)
        for k in range(CW // GCH):
            sl = pl.ds(k * GCH, GCH)
            pltpu.async_copy(ap_hbm.at[wgb.at[sl]], rows.at[sl], sem).wait()

        neg = jnp.full((16,), -1e9, F32)

        def selp(i, carry):
            v = i * 16 + iota
            widx = lax.shift_right_logical(v, 3)
            h = lax.bitwise_and(v, 7)
            wv = plsc.load_gather(winb, [widx])
            rsub = lax.bitwise_and(jnp.maximum(wv, 0), 15)
            av = plsc.load_gather(rows, [widx, rsub * 8 + h])
            aout[pl.ds(i * 16, 16)] = jnp.where(wv >= 0, av, neg)
            return carry

        lax.fori_loop(0, (CW * H) // 16, selp, 0)
        pltpu.sync_copy(aout, af_out.at[pl.ds(wbase * H, CW * H)])

    return select_k


def _make_scatter(E, D, NP, SCH, NB):
    EW = E // NW
    RPW = NP // NS
    NSC = EW // SCH
    NGR = NSC // NB
    mesh = plsc.VectorSubcoreMesh(core_axis_name="c", subcore_axis_name="s")

    @functools.partial(
        pl.kernel,
        out_type=jax.ShapeDtypeStruct((NC, NP, D), F32),
        mesh=mesh,
        compiler_params=_SC_PARAMS,
        scratch_types=[
            pltpu.VMEM((NB, SCH), I32),
            pltpu.VMEM((NB, SCH, D), F32),
            pltpu.VMEM_SHARED((NP, D), F32),
        ] + [pltpu.SemaphoreType.DMA] * NB,
    )
    def scatter_k(dst_hbm, m_hbm, zeros_hbm, msum2, idxb, mrows, msum_sh,
                  *lsem):
        c = lax.axis_index("c")
        sid = lax.axis_index("s")
        base = (c * NS + sid) * EW

        pltpu.sync_copy(zeros_hbm.at[pl.ds(sid * RPW, RPW)],
                        msum_sh.at[pl.ds(sid * RPW, RPW)])
        plsc.subcore_barrier()

        def load(i, b):
            off = base + i * SCH
            pltpu.async_copy(dst_hbm.at[pl.ds(off, SCH)], idxb.at[b],
                             lsem[b])
            pltpu.async_copy(m_hbm.at[pl.ds(off, SCH)], mrows.at[b],
                             lsem[b])

        for b in range(NB):
            load(b, b)

        def body(g, carry):
            for b in range(NB):
                i = g * NB + b
                off = base + i * SCH
                pltpu.make_async_copy(dst_hbm.at[pl.ds(off, SCH)],
                                      idxb.at[b], lsem[b]).wait()
                pltpu.make_async_copy(m_hbm.at[pl.ds(off, SCH)],
                                      mrows.at[b], lsem[b]).wait()
                pltpu.sync_copy(mrows.at[b], msum_sh.at[idxb.at[b]],
                                add=True)

                @pl.when(g < NGR - 1)
                def _():
                    load(i + NB, b)

            return carry

        lax.fori_loop(0, NGR, body, 0)
        plsc.subcore_barrier()
        pltpu.sync_copy(msum_sh.at[pl.ds(sid * RPW, RPW)],
                        msum2.at[c, pl.ds(sid * RPW, RPW)])

    return scatter_k


def _soft_body(a_ref, p_ref):
    a = a_ref[...]
    mx = jnp.max(a, axis=0, keepdims=True)
    e = jnp.exp(a - mx)
    p_ref[...] = e / jnp.sum(e, axis=0, keepdims=True)


def _out_body(x_ref, m0_ref, m1_ref, p_ref, wo1x, wo1a,
              bo1, wo2, bo2, lng, lnb, rep, y_ref):
    x = x_ref[...]
    pfull = jnp.dot(p_ref[...], rep[...], preferred_element_type=F32)
    agg = (m0_ref[...] + m1_ref[...]) * pfull
    h1 = (jnp.dot(x, wo1x[...], preferred_element_type=F32)
          + jnp.dot(agg, wo1a[...], preferred_element_type=F32) + bo1[...])
    h = jnp.dot(_gelu(h1), wo2[...], preferred_element_type=F32) + bo2[...]
    y = x + h
    mu = jnp.mean(y, axis=1, keepdims=True)
    var = jnp.mean((y - mu) ** 2, axis=1, keepdims=True)
    y_ref[...] = (y - mu) / jnp.sqrt(var + 1e-5) * lng[...] + lnb[...]


def _make_out(N, D, H, BN):
    full = lambda shape: pl.BlockSpec(shape, lambda i: (0, 0))
    return pl.pallas_call(
        _out_body,
        grid=(N // BN,),
        in_specs=[
            pl.BlockSpec((BN, D), lambda i: (i, 0)),
            pl.BlockSpec((BN, D), lambda i: (i, 0)),
            pl.BlockSpec((BN, D), lambda i: (i, 0)),
            pl.BlockSpec((BN, H), lambda i: (i, 0)),
            full((D, D)), full((D, D)), full((1, D)),
            full((D, D)), full((1, D)),
            full((1, D)), full((1, D)),
            full((H, D)),
        ],
        out_specs=pl.BlockSpec((BN, D), lambda i: (i, 0)),
        out_shape=jax.ShapeDtypeStruct((N, D), F32),
    )


def kernel(node_features, edge_index, edge_attr, W_msg1, b_msg1, W_msg2,
           b_msg2, W_q, b_q, W_kv, b_kv, W_out1, b_out1, W_out2, b_out2,
           ln_g, ln_b):
    x = node_features
    N, D = x.shape
    E = edge_index.shape[1]
    ED = edge_attr.shape[1]
    H = 8
    HD = D // H
    NP = ((N + 32 * 8 - 1) // (32 * 8)) * (32 * 8)

    src = edge_index[0]
    dst = edge_index[1]

    w1s = W_msg1[:D]
    w1d = W_msg1[D:2 * D]
    w1e = W_msg1[2 * D:]
    wkvs = W_kv[:D]
    wkve = W_kv[D:]
    wo1x = W_out1[:D]
    wo1a = W_out1[D:]
    b1 = b_msg1.reshape(1, D)
    b2 = b_msg2.reshape(1, D)
    bq = b_q.reshape(1, D)
    bkv = b_kv.reshape(1, D)
    bo1 = b_out1.reshape(1, D)
    bo2 = b_out2.reshape(1, D)
    lng = ln_g.reshape(1, D)
    lnb = ln_b.reshape(1, D)

    head_of = jnp.arange(D, dtype=I32) // HD
    sel = (head_of[:, None] == jnp.arange(H, dtype=I32)[None, :]
           ).astype(F32) * (HD ** -0.5)
    selbig = jnp.kron(jnp.eye(16, dtype=F32), sel).astype(BF16)
    rep = (jnp.arange(H, dtype=I32)[:, None] == head_of[None, :]).astype(F32)
    zeros = jnp.zeros((NP, D), F32)

    win, wing = _make_win(E, NP)(dst)
    s_g, d_g = _make_gather(N, E, D, 40, 2)(x, src, dst)
    m, ap = _make_edge(E, D, ED, H, 2560)(
        s_g, d_g, edge_attr, w1s.astype(BF16), w1d.astype(BF16),
        w1e.astype(BF16), b1, W_msg2, b2, W_q.astype(BF16), bq,
        wkvs.astype(BF16), wkve.astype(BF16), bkv, selbig)
    msA = _make_scatter(E, D, NP, 40, 5)(dst, m, zeros)
    af_flat = _make_select(E, H, NP, 80)(win, wing, ap)
    p = pl.pallas_call(
        _soft_body, out_shape=jax.ShapeDtypeStruct((NP, H), F32))(
            af_flat.reshape(NP, H))
    y = _make_out(N, D, H, 1000)(
        x, msA[0], msA[1], p, wo1x, wo1a, bo1, W_out2, bo2,
        lng, lnb, rep)
    return y
